# Initial kernel scaffold; baseline (speedup 1.0000x reference)
import jax
import jax.numpy as jnp
from jax.experimental import pallas as pl


def kernel(z_nodes, t, cart_coords, batch_indices, species, params):
    raise NotImplementedError("write your pallas kernel here")


# speedup vs baseline: 9.4428x; 9.4428x over previous
import functools
import math

import jax
import jax.numpy as jnp
from jax import lax
from jax.experimental import pallas as pl
from jax.experimental.pallas import tpu as pltpu
from jax.experimental.pallas import tpu_sc as plsc

_INTERPRET = False

F32 = jnp.float32
I32 = jnp.int32


def _silu(x):
    return x * jax.nn.sigmoid(x)


@functools.partial(jax.jit, static_argnames=("chunk", "n_workers"))
def _sc_gather(table, idx, chunk, n_workers):
    rows = idx.shape[0]
    d = table.shape[1]
    per_w = rows // n_workers
    n_chunks = per_w // chunk
    mesh = plsc.VectorSubcoreMesh(core_axis_name="c", subcore_axis_name="s")

    @functools.partial(
        pl.kernel,
        out_type=jax.ShapeDtypeStruct((rows, d), table.dtype),
        mesh=mesh,
        scratch_types=[
            pltpu.VMEM((chunk,), I32),
            pltpu.VMEM((chunk, d), table.dtype),
            pltpu.SemaphoreType.DMA,
        ],
    )
    def gather_kernel(table_hbm, idx_hbm, out_hbm, idx_v, rows_v, sem):
        wid = lax.axis_index("s") * 2 + lax.axis_index("c")

        @pl.when(wid < n_workers)
        def _():
            base = wid * per_w
            for c in range(n_chunks):
                off = base + c * chunk
                pltpu.sync_copy(idx_hbm.at[pl.ds(off, chunk)], idx_v)
                pltpu.async_copy(table_hbm.at[idx_v], rows_v, sem).wait()
                pltpu.sync_copy(rows_v, out_hbm.at[pl.ds(off, chunk)])

    return gather_kernel(table, idx)


def _prep_body(z_ref, sp_ref, b_ref, t_ref, sptab_ref, tw1_ref, tb1_ref,
               tw2_ref, tb2_ref, nwz_ref, nws_ref, nb_ref, w1a_ref, w1b_ref,
               w1d_ref, b1_ref, h_out, tn_out, a_out, bv_out, *, n_batch,
               time_dim):
    half = time_dim // 2
    tcol = t_ref[...]
    ilane = lax.broadcasted_iota(I32, (1, half), 1).astype(F32)
    freqs = jnp.exp(ilane * F32(-(math.log(10000.0) / (half - 1))))
    a = tcol * freqs
    emb = jnp.concatenate([jnp.sin(a), jnp.cos(a)], axis=1)
    e1 = _silu(jnp.dot(emb, tw1_ref[...], preferred_element_type=F32)
               + tb1_ref[...])
    temb = jnp.dot(e1, tw2_ref[...], preferred_element_type=F32) + tb2_ref[...]
    pad_rows = 128 - n_batch
    temb_pad = jnp.concatenate(
        [temb, jnp.zeros((pad_rows, time_dim), F32)], axis=0)

    lane128 = lax.broadcasted_iota(I32, (1, 128), 1)
    b_oh = (b_ref[...] == lane128).astype(F32)
    tn = jnp.dot(b_oh, temb_pad, preferred_element_type=F32)

    s_oh = (sp_ref[...] == lane128).astype(F32)
    spemb = jnp.dot(s_oh, sptab_ref[...], preferred_element_type=F32)

    h0 = (jnp.dot(z_ref[...], nwz_ref[...], preferred_element_type=F32)
          + jnp.dot(spemb, nws_ref[...], preferred_element_type=F32)
          + nb_ref[...])

    h_out[...] = h0
    tn_out[...] = tn
    a_out[...] = jnp.dot(h0, w1a_ref[...], preferred_element_type=F32)
    bv_out[...] = (jnp.dot(h0, w1b_ref[...], preferred_element_type=F32)
                   + jnp.dot(tn, w1d_ref[...], preferred_element_type=F32)
                   + b1_ref[...])


def _knn_body(xq_ref, yq_ref, zq_ref, bq_ref, xc_ref, yc_ref, zc_ref, bc_ref,
              lo_ref, hi_ref, nbr_out, w_out, *, rows, cols, k, kp):
    g = pl.program_id(0)
    r0 = g * rows
    xi = xq_ref[0]
    yi = yq_ref[0]
    zi = zq_ref[0]
    bi = bq_ref[0]
    sqi = xi * xi + yi * yi + zi * zi
    xib = xi.astype(jnp.bfloat16).astype(F32)
    yib = yi.astype(jnp.bfloat16).astype(F32)
    zib = zi.astype(jnp.bfloat16).astype(F32)
    rid = r0 + lax.broadcasted_iota(I32, (1, rows), 1)
    lo = lo_ref[g]
    hi = hi_ref[g]
    base = (lo // cols) * cols
    n_chunk = (hi - base + cols - 1) // cols
    inf = F32(jnp.inf)
    bigi = I32(2**31 - 1)

    def chunk_body(c, carry):
        vals, idxs = carry
        c0 = base + c * cols
        col_id = c0 + lax.broadcasted_iota(I32, (cols, 1), 0)
        xj = xc_ref[pl.ds(c0, cols), 0:1]
        yj = yc_ref[pl.ds(c0, cols), 0:1]
        zj = zc_ref[pl.ds(c0, cols), 0:1]
        bj = bc_ref[pl.ds(c0, cols), 0:1]
        sqj = xj * xj + yj * yj + zj * zj
        xjb = xj.astype(jnp.bfloat16).astype(F32)
        yjb = yj.astype(jnp.bfloat16).astype(F32)
        zjb = zj.astype(jnp.bfloat16).astype(F32)
        dot = xjb * xib + yjb * yib + zjb * zib
        d2 = jnp.maximum(sqi + sqj - 2.0 * dot, 0.0)
        valid = (bj == bi) & (col_id != rid)
        d2 = jnp.where(valid, d2, inf)
        cand = jnp.concatenate([vals, d2], axis=0)
        cidx = jnp.concatenate(
            [idxs, jnp.broadcast_to(col_id, (cols, rows))], axis=0)
        new_v, new_i = [], []
        for _ in range(k):
            m = jnp.min(cand, axis=0, keepdims=True)
            sel = jnp.where(cand == m, cidx, bigi)
            jm = jnp.min(sel, axis=0, keepdims=True)
            cand = jnp.where(cidx == jm, inf, cand)
            new_v.append(m)
            new_i.append(jm)
        vals = jnp.concatenate(
            new_v + [jnp.full((kp - k, rows), inf, F32)], axis=0)
        idxs = jnp.concatenate(
            new_i + [jnp.zeros((kp - k, rows), I32)], axis=0)
        return vals, idxs

    vals0 = jnp.full((kp, rows), inf, F32)
    idxs0 = jnp.zeros((kp, rows), I32)
    vals, idxs = lax.fori_loop(0, n_chunk, chunk_body, (vals0, idxs0))
    nbr_out[0] = idxs
    w_out[0] = jnp.where(vals < inf, F32(1.0), F32(0.0))


def _knn_call(xq, yq, zq, bq, xc, yc, zc, bc, lo_b, hi_b, *, n_tiles, tile,
              cols, k, kp):
    qspec = lambda: pl.BlockSpec((1, 1, tile), lambda i: (i, 0, 0))
    tspec = lambda: pl.BlockSpec((1, kp, tile), lambda i: (i, 0, 0))
    full = lambda arr: pl.BlockSpec(arr.shape, lambda i: (0,) * arr.ndim)
    knn = pl.pallas_call(
        functools.partial(_knn_body, rows=tile, cols=cols, k=k, kp=kp),
        grid=(n_tiles,),
        in_specs=[
            qspec(), qspec(), qspec(), qspec(),
            full(xc), full(yc), full(zc), full(bc),
            pl.BlockSpec(memory_space=pltpu.SMEM),
            pl.BlockSpec(memory_space=pltpu.SMEM),
        ],
        out_specs=[tspec(), tspec()],
        out_shape=[jax.ShapeDtypeStruct((n_tiles, kp, tile), I32),
                   jax.ShapeDtypeStruct((n_tiles, kp, tile), jnp.float32)],
        interpret=_INTERPRET,
    )
    return knn(xq, yq, zq, bq, xc, yc, zc, bc, lo_b, hi_b)


def _layer_body(*refs, k, last):
    (h_ref, bv_ref, tn_ref, w_ref, cd_ref, g_ref, cg_ref,
     w1c_ref, w2_ref, b2_ref, cw1_ref, cb1_ref, cw2_ref,
     nw1a_ref, nw1b_ref, nw1c_ref, nb1_ref, nw2_ref, nb2_ref) = refs[:19]
    if last:
        h_out, coord_out = refs[19:]
    else:
        w1a_ref, w1b_ref, w1d_ref, b1_ref = refs[19:23]
        h_out, coord_out, a_out, bv_out = refs[23:]

    h = h_ref[...]
    bv = bv_ref[...]
    tn = tn_ref[...]
    w = w_ref[...]
    cd = cd_ref[...]
    cdx = cd[:, 0:1]
    cdy = cd[:, 1:2]
    cdz = cd[:, 2:3]
    w1c = w1c_ref[...]
    w2 = w2_ref[...]
    b2 = b2_ref[...]
    cw1 = cw1_ref[...]
    cb1 = cb1_ref[...]
    cw2 = cw2_ref[...]

    rows = h.shape[0]
    m_acc = jnp.zeros((rows, 128), F32)
    cx = jnp.zeros((rows, 1), F32)
    cy = jnp.zeros((rows, 1), F32)
    cz = jnp.zeros((rows, 1), F32)
    for kk in range(k):
        gk = g_ref[kk]
        ck = cg_ref[kk]
        dx = ck[:, 0:1] - cdx
        dy = ck[:, 1:2] - cdy
        dz = ck[:, 2:3] - cdz
        dsq = dx * dx + dy * dy + dz * dz
        wk = w[:, kk:kk + 1]
        m1 = _silu(gk + bv + dsq * w1c)
        mij = _silu(jnp.dot(m1, w2, preferred_element_type=F32) + b2)
        c1 = _silu(jnp.dot(mij, cw1, preferred_element_type=F32) + cb1)
        csc = jnp.sum(c1 * cw2, axis=1, keepdims=True)
        dist = jnp.sqrt(dsq + F32(1e-8))
        cf = csc * wk / dist
        cx = cx + dx * cf
        cy = cy + dy * cf
        cz = cz + dz * cf
        m_acc = m_acc + mij * wk

    hu = _silu(jnp.dot(h, nw1a_ref[...], preferred_element_type=F32)
               + jnp.dot(m_acc, nw1b_ref[...], preferred_element_type=F32)
               + jnp.dot(tn, nw1c_ref[...], preferred_element_type=F32)
               + nb1_ref[...])
    h_new = h + jnp.dot(hu, nw2_ref[...], preferred_element_type=F32) \
        + nb2_ref[...]
    h_out[...] = h_new
    coord_out[...] = jnp.concatenate(
        [cx, cy, cz, jnp.zeros((rows, 1), F32)], axis=1)
    if not last:
        a_out[...] = jnp.dot(h_new, w1a_ref[...], preferred_element_type=F32)
        bv_out[...] = (jnp.dot(h_new, w1b_ref[...], preferred_element_type=F32)
                       + jnp.dot(tn, w1d_ref[...], preferred_element_type=F32)
                       + b1_ref[...])


def kernel(z_nodes, t, cart_coords, batch_indices, species, params):
    n = z_nodes.shape[0]
    n_batch = t.shape[0]
    d = z_nodes.shape[1]
    time_dim = params['time_w2'].shape[1]
    k = min(12, n // n_batch - 1)
    kp = 16
    tile = 400
    n_tiles = n // tile
    cols = 512
    n_pad = ((n + cols + 127) // 128) * 128

    f32 = F32
    coords = cart_coords.astype(f32)
    batch_i32 = batch_indices.astype(I32)

    xq = coords[:, 0].reshape(n_tiles, 1, tile)
    yq = coords[:, 1].reshape(n_tiles, 1, tile)
    zq = coords[:, 2].reshape(n_tiles, 1, tile)
    bq = batch_i32.reshape(n_tiles, 1, tile)
    xc = jnp.pad(coords[:, 0], (0, n_pad - n))[:, None]
    yc = jnp.pad(coords[:, 1], (0, n_pad - n))[:, None]
    zc = jnp.pad(coords[:, 2], (0, n_pad - n))[:, None]
    bc = jnp.pad(batch_i32, (0, n_pad - n), constant_values=-1)[:, None]
    b_col = batch_i32[:, None]
    sp_col = species.astype(I32)[:, None]
    coords16 = jnp.pad(coords, ((0, 0), (0, 13)))

    r0s = jnp.arange(n_tiles) * tile
    first_b = batch_i32[r0s]
    last_b = batch_i32[jnp.minimum(r0s + tile - 1, n - 1)]
    lo_b = jnp.searchsorted(batch_i32, first_b, side='left').astype(I32)
    hi_b = jnp.searchsorted(batch_i32, last_b, side='right').astype(I32)

    p = params
    sptab = jnp.pad(p['species_table'],
                    ((0, 128 - p['species_table'].shape[0]), (0, 0)))
    node_w = p['node_w']
    nwz = node_w[:d]
    nws = node_w[d:]
    layers = p['layers']

    def split_edge_w1(lp):
        w1 = lp['edge_w1']
        return (w1[:128], w1[128:256], w1[256:257], w1[257:])

    w1a0, w1b0, w1c0, w1d0 = split_edge_w1(layers[0])

    row = lambda v: v[None, :]

    grid = (n_tiles,)
    bspec = lambda bs, im: pl.BlockSpec(bs, im)
    full = lambda arr: pl.BlockSpec(arr.shape, lambda i: (0,) * arr.ndim)
    tile_spec = lambda w: pl.BlockSpec((tile, w), lambda i: (i, 0))

    prep = pl.pallas_call(
        functools.partial(_prep_body, n_batch=n_batch, time_dim=time_dim),
        grid=grid,
        in_specs=[
            tile_spec(d), tile_spec(1), tile_spec(1),
            full(t[:, None]), full(sptab),
            full(p['time_w1']), full(row(p['time_b1'])),
            full(p['time_w2']), full(row(p['time_b2'])),
            full(nwz), full(nws), full(row(p['node_b'])),
            full(w1a0), full(w1b0), full(w1d0),
            full(row(layers[0]['edge_b1'])),
        ],
        out_specs=[tile_spec(128), tile_spec(time_dim),
                   tile_spec(128), tile_spec(128)],
        out_shape=[jax.ShapeDtypeStruct((n, 128), f32),
                   jax.ShapeDtypeStruct((n, time_dim), f32),
                   jax.ShapeDtypeStruct((n, 128), f32),
                   jax.ShapeDtypeStruct((n, 128), f32)],
        interpret=_INTERPRET,
    )
    h, tn, a_cur, bv_cur = prep(
        z_nodes.astype(f32), sp_col, b_col, t.astype(f32)[:, None], sptab,
        p['time_w1'], row(p['time_b1']), p['time_w2'], row(p['time_b2']),
        nwz, nws, row(p['node_b']), w1a0, w1b0, w1d0,
        row(layers[0]['edge_b1']))

    nbr_t3, wgt_t3 = _knn_call(xq, yq, zq, bq, xc, yc, zc, bc, lo_b, hi_b,
                               n_tiles=n_tiles, tile=tile, cols=cols,
                               k=k, kp=kp)
    wgt = wgt_t3.transpose(0, 2, 1).reshape(n, kp)

    src_flat = nbr_t3[:, :k, :].transpose(1, 0, 2).reshape(k * n)
    src_flat = jnp.clip(src_flat, 0, n - 1)
    coords_tab = jnp.pad(coords, ((0, 0), (0, 128 - coords.shape[1])))
    cg = _sc_gather(coords_tab, src_flat, chunk=400, n_workers=30)
    cg = cg[:, :16].reshape(k, n, 16)

    total = jnp.zeros((n, 4), f32)
    for li, lp in enumerate(layers):
        last = li == len(layers) - 1
        g3 = _sc_gather(a_cur, src_flat, chunk=400, n_workers=30)
        g3 = g3.reshape(k, n, 128)

        in_specs = [
            tile_spec(128), tile_spec(128), tile_spec(time_dim),
            tile_spec(kp), tile_spec(16),
            pl.BlockSpec((k, tile, 128), lambda i: (0, i, 0)),
            pl.BlockSpec((k, tile, 16), lambda i: (0, i, 0)),
        ]
        _, _, w1c_l, _ = split_edge_w1(lp)
        args = [h, bv_cur, tn, wgt, coords16, g3, cg,
                row(w1c_l[0]), lp['edge_w2'], row(lp['edge_b2']),
                lp['coord_w1'], row(lp['coord_b1']), row(lp['coord_w2'][:, 0]),
                lp['node_w1'][:128], lp['node_w1'][128:256],
                lp['node_w1'][256:], row(lp['node_b1']),
                lp['node_w2'], row(lp['node_b2'])]
        in_specs += [full(a) for a in args[7:]]
        out_specs = [tile_spec(128), tile_spec(4)]
        out_shape = [jax.ShapeDtypeStruct((n, 128), f32),
                     jax.ShapeDtypeStruct((n, 4), f32)]
        if not last:
            nxt = layers[li + 1]
            w1a_n, w1b_n, _, w1d_n = split_edge_w1(nxt)
            extra = [w1a_n, w1b_n, w1d_n, row(nxt['edge_b1'])]
            args += extra
            in_specs += [full(a) for a in extra]
            out_specs += [tile_spec(128), tile_spec(128)]
            out_shape += [jax.ShapeDtypeStruct((n, 128), f32),
                          jax.ShapeDtypeStruct((n, 128), f32)]

        layer_call = pl.pallas_call(
            functools.partial(_layer_body, k=k, last=last),
            grid=grid,
            in_specs=in_specs,
            out_specs=out_specs,
            out_shape=out_shape,
            interpret=_INTERPRET,
        )
        outs = layer_call(*args)
        if last:
            h, coord_d = outs
        else:
            h, coord_d, a_cur, bv_cur = outs
        total = total + coord_d

    return total[:, :3], h
